# baseline (speedup 1.0000x reference)
import functools

import jax
import jax.numpy as jnp
from jax import lax
from jax.experimental import pallas as pl
from jax.experimental.pallas import tpu as pltpu
from jax.experimental.pallas import tpu_sc as plsc

_L = 200
_D = 64
_LANES = 16
_SEQ_PER_CHUNK = 2
_CL = _SEQ_PER_CHUNK * _L
_NBUF = 3
_DEPTH = 2


def _emb_body(tok_hbm, idx_hbm, pos_hbm, out_hbm,
              idx_all, pos_v, b0, b1, b2,
              g0, g1, g2, o0, o1, o2,
              *, seqs_per_w, num_cores):
    bufs = (b0, b1, b2)
    gsems = (g0, g1, g2)
    osems = (o0, o1, o2)
    nrows = seqs_per_w * _L
    nchunks = nrows // _CL
    wid = lax.axis_index("s") * num_cores + lax.axis_index("c")
    base = pl.multiple_of(wid * nrows, _CL)

    pltpu.sync_copy(pos_hbm, pos_v)
    pltpu.sync_copy(idx_hbm.at[pl.ds(base, nrows)], idx_all)

    def start_gather(c, b):
        off = pl.multiple_of(c * _CL, 8)
        pltpu.async_copy(tok_hbm.at[idx_all.at[pl.ds(off, _CL)]],
                         bufs[b], gsems[b])

    def drain_gather(b):
        pltpu.make_async_copy(tok_hbm.at[pl.ds(0, _CL)], bufs[b],
                              gsems[b]).wait()

    def start_out(c, b):
        off = pl.multiple_of(base + c * _CL, _CL)
        pltpu.async_copy(bufs[b],
                         out_hbm.at[pl.ds(off, _CL), pl.ds(0, _D)],
                         osems[b])

    def drain_out(b):
        pltpu.make_async_copy(bufs[b], out_hbm.at[pl.ds(0, _CL), pl.ds(0, _D)],
                              osems[b]).wait()

    def add_pos(b):
        def add_row(l, c2):
            buf = bufs[b]
            for k in range(_D // _LANES):
                sl = pl.ds(k * _LANES, _LANES)
                p = pos_v[l, sl]
                for s in range(_SEQ_PER_CHUNK):
                    r = s * _L + l
                    buf[r, sl] = buf[r, sl] + p
            return c2
        lax.fori_loop(0, _L, add_row, 0, unroll=2)

    def process(c, b):
        drain_gather(b)
        add_pos(b)
        start_out(c, b)
        b2_ = (b + _DEPTH) % _NBUF

        @pl.when(c + _DEPTH < nchunks)
        def _():
            @pl.when(c >= 1)
            def _():
                drain_out(b2_)
            start_gather(c + _DEPTH, b2_)

    for b in range(_DEPTH):
        start_gather(b, b)

    def wave(w, carry):
        for b in range(_NBUF):
            process(w * _NBUF + b, b)
        return carry

    lax.fori_loop(0, (nchunks - 1) // _NBUF, wave, 0)
    process(nchunks - 1, (nchunks - 1) % _NBUF)
    for b in range(_NBUF):
        drain_out(b)


_TBLK = 512


def _tpad_body(tT_ref, out_ref):
    row = lax.broadcasted_iota(jnp.int32, (_D, _D), 0)
    col = lax.broadcasted_iota(jnp.int32, (_D, _D), 1)
    eye = jnp.where(row == col, 1.0, 0.0).astype(jnp.float32)
    blk = tT_ref[...]
    out_ref[:, :_D] = lax.dot_general(blk, eye, (((0,), (0,)), ((), ())))


def kernel(x, token_table, pos_table):
    B, L = x.shape
    V, D = token_table.shape
    assert L == _L and D == _D
    info = plsc.get_sparse_core_info()
    nw = info.num_cores * info.num_subcores
    assert B % nw == 0
    seqs_per_w = B // nw

    mesh = plsc.VectorSubcoreMesh(core_axis_name="c", subcore_axis_name="s")
    nblk = (V + _TBLK - 1) // _TBLK
    tok_pad = pl.pallas_call(
        _tpad_body,
        grid=(nblk,),
        in_specs=[pl.BlockSpec((_D, _TBLK), lambda i: (0, i))],
        out_specs=pl.BlockSpec((_TBLK, 2 * _D), lambda i: (i, 0)),
        out_shape=jax.ShapeDtypeStruct((V, 2 * D), jnp.float32),
    )(token_table.T)
    tok2 = tok_pad.reshape(2 * V, D)
    flat_idx = x.reshape(B * L).astype(jnp.int32) * 2

    sems = [pltpu.SemaphoreType.DMA] * (2 * _NBUF)
    emb = functools.partial(
        pl.kernel,
        mesh=mesh,
        out_type=jax.ShapeDtypeStruct((B * L, 2 * D), jnp.float32),
        scratch_types=[
            pltpu.VMEM((seqs_per_w * _L,), jnp.int32),
            pltpu.VMEM((_L, _D), jnp.float32),
        ] + [pltpu.VMEM((_CL, _D), jnp.float32) for _ in range(_NBUF)]
          + sems,
        compiler_params=pltpu.CompilerParams(use_tc_tiling_on_sc=False),
    )(functools.partial(_emb_body, seqs_per_w=seqs_per_w,
                        num_cores=info.num_cores))

    out = emb(tok2, flat_idx, pos_table)
    return out[:, :_D].reshape(B, L, D)


# speedup vs baseline: 1.5883x; 1.5883x over previous
import functools

import jax
import jax.numpy as jnp
from jax import lax
from jax.experimental import pallas as pl
from jax.experimental.pallas import tpu as pltpu
from jax.experimental.pallas import tpu_sc as plsc

_L = 200
_D = 64
_LANES = 16
_SEQ_PER_CHUNK = 2
_CL = _SEQ_PER_CHUNK * _L
_NBUF = 3
_DEPTH = 2


def _emb_body(tok_hbm, idx_hbm, pos_hbm, out_hbm,
              idx_all, pos_v, b0, b1, b2,
              g0, g1, g2, o0, o1, o2,
              *, seqs_per_w, num_cores):
    bufs = (b0, b1, b2)
    gsems = (g0, g1, g2)
    osems = (o0, o1, o2)
    nrows = seqs_per_w * _L
    nchunks = nrows // _CL
    wid = lax.axis_index("s") * num_cores + lax.axis_index("c")
    base = pl.multiple_of(wid * nrows, _CL)

    pltpu.sync_copy(pos_hbm, pos_v)
    pltpu.sync_copy(idx_hbm.at[pl.ds(base, nrows)], idx_all)

    def start_gather(c, b):
        off = pl.multiple_of(c * _CL, 8)
        pltpu.async_copy(tok_hbm.at[idx_all.at[pl.ds(off, _CL)]],
                         bufs[b], gsems[b])

    def drain_gather(b):
        pltpu.make_async_copy(tok_hbm.at[pl.ds(0, _CL)], bufs[b],
                              gsems[b]).wait()

    def start_out(c, b):
        off = pl.multiple_of(base + c * _CL, _CL)
        pltpu.async_copy(bufs[b],
                         out_hbm.at[pl.ds(off, _CL), pl.ds(0, _D)],
                         osems[b])

    def drain_out(b):
        pltpu.make_async_copy(bufs[b], out_hbm.at[pl.ds(0, _CL), pl.ds(0, _D)],
                              osems[b]).wait()

    def add_pos(b):
        def add_row(l, c2):
            buf = bufs[b]
            for k in range(_D // _LANES):
                sl = pl.ds(k * _LANES, _LANES)
                p = pos_v[l, sl]
                for s in range(_SEQ_PER_CHUNK):
                    r = s * _L + l
                    buf[r, sl] = buf[r, sl] + p
            return c2
        lax.fori_loop(0, _L, add_row, 0, unroll=2)

    def process(c, b):
        drain_gather(b)
        add_pos(b)
        start_out(c, b)
        b2_ = (b + _DEPTH) % _NBUF

        @pl.when(c + _DEPTH < nchunks)
        def _():
            @pl.when(c >= 1)
            def _():
                drain_out(b2_)
            start_gather(c + _DEPTH, b2_)

    for b in range(_DEPTH):
        start_gather(b, b)

    def wave(w, carry):
        for b in range(_NBUF):
            process(w * _NBUF + b, b)
        return carry

    lax.fori_loop(0, (nchunks - 1) // _NBUF, wave, 0)
    process(nchunks - 1, (nchunks - 1) % _NBUF)
    for b in range(_NBUF):
        drain_out(b)


_TBLK = 2048


def _tpad_body(tT_ref, out_ref):
    row = lax.broadcasted_iota(jnp.int32, (_D, _D), 0)
    col = lax.broadcasted_iota(jnp.int32, (_D, _D), 1)
    eye = jnp.where(row == col, 1.0, 0.0).astype(jnp.float32)
    blk = tT_ref[...]
    r = lax.dot_general(blk, eye, (((0,), (0,)), ((), ())),
                        precision=lax.Precision.HIGHEST)
    out_ref[...] = jnp.concatenate([r, r], axis=1)


def kernel(x, token_table, pos_table):
    B, L = x.shape
    V, D = token_table.shape
    assert L == _L and D == _D
    info = plsc.get_sparse_core_info()
    nw = info.num_cores * info.num_subcores
    assert B % nw == 0
    seqs_per_w = B // nw

    mesh = plsc.VectorSubcoreMesh(core_axis_name="c", subcore_axis_name="s")
    nblk = (V + _TBLK - 1) // _TBLK
    tok_pad = pl.pallas_call(
        _tpad_body,
        grid=(nblk,),
        in_specs=[pl.BlockSpec((_D, _TBLK), lambda i: (0, i))],
        out_specs=pl.BlockSpec((_TBLK, 2 * _D), lambda i: (i, 0)),
        out_shape=jax.ShapeDtypeStruct((V, 2 * D), jnp.float32),
    )(token_table.T)
    tok2 = tok_pad.reshape(2 * V, D)
    flat_idx = x.reshape(B * L).astype(jnp.int32) * 2

    sems = [pltpu.SemaphoreType.DMA] * (2 * _NBUF)
    emb = functools.partial(
        pl.kernel,
        mesh=mesh,
        out_type=jax.ShapeDtypeStruct((B * L, 2 * D), jnp.float32),
        scratch_types=[
            pltpu.VMEM((seqs_per_w * _L,), jnp.int32),
            pltpu.VMEM((_L, _D), jnp.float32),
        ] + [pltpu.VMEM((_CL, _D), jnp.float32) for _ in range(_NBUF)]
          + sems,
        compiler_params=pltpu.CompilerParams(use_tc_tiling_on_sc=False),
    )(functools.partial(_emb_body, seqs_per_w=seqs_per_w,
                        num_cores=info.num_cores))

    out = emb(tok2, flat_idx, pos_table)
    return out[:, :_D].reshape(B, L, D)


# speedup vs baseline: 1.7763x; 1.1184x over previous
import functools

import jax
import jax.numpy as jnp
from jax import lax
from jax.experimental import pallas as pl
from jax.experimental.pallas import tpu as pltpu
from jax.experimental.pallas import tpu_sc as plsc

_L = 200
_D = 64
_LANES = 16
_SEQ_PER_CHUNK = 2
_CL = _SEQ_PER_CHUNK * _L
_NBUF = 3
_DEPTH = 2


def _emb_body(tok_hbm, idx_hbm, pos_hbm, out_hbm,
              idx_all, pos_v, b0, b1, b2,
              g0, g1, g2, o0, o1, o2,
              *, seqs_per_w, num_cores):
    bufs = (b0, b1, b2)
    gsems = (g0, g1, g2)
    osems = (o0, o1, o2)
    nrows = seqs_per_w * _L
    nchunks = nrows // _CL
    wid = lax.axis_index("s") * num_cores + lax.axis_index("c")
    base = pl.multiple_of(wid * nrows, _CL)

    pltpu.sync_copy(pos_hbm, pos_v)
    pltpu.sync_copy(idx_hbm.at[pl.ds(base, nrows)], idx_all)

    def start_gather(c, b):
        off = pl.multiple_of(c * _CL, 8)
        pltpu.async_copy(tok_hbm.at[idx_all.at[pl.ds(off, _CL)]],
                         bufs[b], gsems[b])

    def drain_gather(b):
        pltpu.make_async_copy(tok_hbm.at[pl.ds(0, _CL)], bufs[b],
                              gsems[b]).wait()

    def start_out(c, b):
        off = pl.multiple_of(base + c * _CL, _CL)
        pltpu.async_copy(bufs[b],
                         out_hbm.at[pl.ds(off, _CL), pl.ds(0, _D)],
                         osems[b])

    def drain_out(b):
        pltpu.make_async_copy(bufs[b], out_hbm.at[pl.ds(0, _CL), pl.ds(0, _D)],
                              osems[b]).wait()

    def add_pos(b):
        def add_row(l, c2):
            buf = bufs[b]
            for k in range(_D // _LANES):
                sl = pl.ds(k * _LANES, _LANES)
                p = pos_v[l, sl]
                for s in range(_SEQ_PER_CHUNK):
                    r = s * _L + l
                    buf[r, sl] = buf[r, sl] + p
            return c2
        lax.fori_loop(0, _L, add_row, 0, unroll=2)

    def process(c, b):
        drain_gather(b)
        add_pos(b)
        start_out(c, b)
        b2_ = (b + _DEPTH) % _NBUF

        @pl.when(c + _DEPTH < nchunks)
        def _():
            @pl.when(c >= 1)
            def _():
                drain_out(b2_)
            start_gather(c + _DEPTH, b2_)

    for b in range(_DEPTH):
        start_gather(b, b)

    def wave(w, carry):
        for b in range(_NBUF):
            process(w * _NBUF + b, b)
        return carry

    lax.fori_loop(0, (nchunks - 1) // _NBUF, wave, 0)
    process(nchunks - 1, (nchunks - 1) % _NBUF)
    for b in range(_NBUF):
        drain_out(b)


_TBLK = 2048


def _tpad_body(tT_ref, out_ref):
    r = tT_ref[...].T
    out_ref[...] = jnp.concatenate([r, r], axis=1)


def kernel(x, token_table, pos_table):
    B, L = x.shape
    V, D = token_table.shape
    assert L == _L and D == _D
    info = plsc.get_sparse_core_info()
    nw = info.num_cores * info.num_subcores
    assert B % nw == 0
    seqs_per_w = B // nw

    mesh = plsc.VectorSubcoreMesh(core_axis_name="c", subcore_axis_name="s")
    nblk = (V + _TBLK - 1) // _TBLK
    tok_pad = pl.pallas_call(
        _tpad_body,
        grid=(nblk,),
        in_specs=[pl.BlockSpec((_D, _TBLK), lambda i: (0, i))],
        out_specs=pl.BlockSpec((_TBLK, 2 * _D), lambda i: (i, 0)),
        out_shape=jax.ShapeDtypeStruct((V, 2 * D), jnp.float32),
    )(token_table.T)
    tok2 = tok_pad.reshape(2 * V, D)
    flat_idx = x.reshape(B * L).astype(jnp.int32) * 2

    sems = [pltpu.SemaphoreType.DMA] * (2 * _NBUF)
    emb = functools.partial(
        pl.kernel,
        mesh=mesh,
        out_type=jax.ShapeDtypeStruct((B * L, 2 * D), jnp.float32),
        scratch_types=[
            pltpu.VMEM((seqs_per_w * _L,), jnp.int32),
            pltpu.VMEM((_L, _D), jnp.float32),
        ] + [pltpu.VMEM((_CL, _D), jnp.float32) for _ in range(_NBUF)]
          + sems,
        compiler_params=pltpu.CompilerParams(use_tc_tiling_on_sc=False),
    )(functools.partial(_emb_body, seqs_per_w=seqs_per_w,
                        num_cores=info.num_cores))

    out = emb(tok2, flat_idx, pos_table)
    return out[:, :_D].reshape(B, L, D)


# speedup vs baseline: 2.0293x; 1.1425x over previous
import functools

import jax
import jax.numpy as jnp
from jax import lax
from jax.experimental import pallas as pl
from jax.experimental.pallas import tpu as pltpu
from jax.experimental.pallas import tpu_sc as plsc

_L = 200
_D = 64
_LANES = 16
_SEQ_PER_CHUNK = 2
_CL = _SEQ_PER_CHUNK * _L
_NBUF = 3
_DEPTH = 2


def _emb_body(tok_hbm, idx_hbm, pos_hbm, out_hbm,
              idx_all, pos_v, b0, b1, b2,
              g0, g1, g2, o0, o1, o2,
              *, seqs_per_w, num_cores):
    bufs = (b0, b1, b2)
    gsems = (g0, g1, g2)
    osems = (o0, o1, o2)
    nrows = seqs_per_w * _L
    nchunks = nrows // _CL
    wid = lax.axis_index("s") * num_cores + lax.axis_index("c")
    base = pl.multiple_of(wid * nrows, _CL)

    pltpu.sync_copy(pos_hbm, pos_v)
    pltpu.sync_copy(idx_hbm.at[pl.ds(base, nrows)], idx_all)

    def start_gather(c, b):
        off = pl.multiple_of(c * _CL, 8)
        pltpu.async_copy(tok_hbm.at[idx_all.at[pl.ds(off, _CL)]],
                         bufs[b], gsems[b])

    def drain_gather(b):
        pltpu.make_async_copy(tok_hbm.at[pl.ds(0, _CL)], bufs[b],
                              gsems[b]).wait()

    def start_out(c, b):
        off = pl.multiple_of(base + c * _CL, _CL)
        pltpu.async_copy(bufs[b],
                         out_hbm.at[pl.ds(off, _CL), pl.ds(0, _D)],
                         osems[b])

    def drain_out(b):
        pltpu.make_async_copy(bufs[b], out_hbm.at[pl.ds(0, _CL), pl.ds(0, _D)],
                              osems[b]).wait()

    def add_pos(b):
        def add_row(l, c2):
            buf = bufs[b]
            for k in range(_D // _LANES):
                sl = pl.ds(k * _LANES, _LANES)
                p = pos_v[l, sl]
                for s in range(_SEQ_PER_CHUNK):
                    r = s * _L + l
                    buf[r, sl] = buf[r, sl] + p
            return c2
        lax.fori_loop(0, _L, add_row, 0, unroll=2)

    def process(c, b):
        drain_gather(b)
        add_pos(b)
        start_out(c, b)
        b2_ = (b + _DEPTH) % _NBUF

        @pl.when(c + _DEPTH < nchunks)
        def _():
            @pl.when(c >= 1)
            def _():
                drain_out(b2_)
            start_gather(c + _DEPTH, b2_)

    for b in range(_DEPTH):
        start_gather(b, b)

    def wave(w, carry):
        for b in range(_NBUF):
            process(w * _NBUF + b, b)
        return carry

    lax.fori_loop(0, (nchunks - 1) // _NBUF, wave, 0)
    process(nchunks - 1, (nchunks - 1) % _NBUF)
    for b in range(_NBUF):
        drain_out(b)


_TBLK = 4096


def _tpad_body(tT_ref, out_ref):
    r = tT_ref[...].T
    out_ref[...] = jnp.concatenate([r, r], axis=1)


def kernel(x, token_table, pos_table):
    B, L = x.shape
    V, D = token_table.shape
    assert L == _L and D == _D
    info = plsc.get_sparse_core_info()
    nw = info.num_cores * info.num_subcores
    assert B % nw == 0
    seqs_per_w = B // nw

    mesh = plsc.VectorSubcoreMesh(core_axis_name="c", subcore_axis_name="s")
    nblk = (V + _TBLK - 1) // _TBLK
    tok_pad = pl.pallas_call(
        _tpad_body,
        grid=(nblk,),
        in_specs=[pl.BlockSpec((_D, _TBLK), lambda i: (0, i))],
        out_specs=pl.BlockSpec((_TBLK, 2 * _D), lambda i: (i, 0)),
        out_shape=jax.ShapeDtypeStruct((V, 2 * D), jnp.float32),
    )(token_table.T)
    tok2 = tok_pad.reshape(2 * V, D)
    flat_idx = x.reshape(B * L).astype(jnp.int32) * 2

    sems = [pltpu.SemaphoreType.DMA] * (2 * _NBUF)
    emb = functools.partial(
        pl.kernel,
        mesh=mesh,
        out_type=jax.ShapeDtypeStruct((B * L, 2 * D), jnp.float32),
        scratch_types=[
            pltpu.VMEM((seqs_per_w * _L,), jnp.int32),
            pltpu.VMEM((_L, _D), jnp.float32),
        ] + [pltpu.VMEM((_CL, _D), jnp.float32) for _ in range(_NBUF)]
          + sems,
        compiler_params=pltpu.CompilerParams(use_tc_tiling_on_sc=False),
    )(functools.partial(_emb_body, seqs_per_w=seqs_per_w,
                        num_cores=info.num_cores))

    out = emb(tok2, flat_idx, pos_table)
    return out[:, :_D].reshape(B, L, D)


# speedup vs baseline: 2.2078x; 1.0879x over previous
import functools

import jax
import jax.numpy as jnp
from jax import lax
from jax.experimental import pallas as pl
from jax.experimental.pallas import tpu as pltpu
from jax.experimental.pallas import tpu_sc as plsc

_L = 200
_D = 64
_LANES = 16
_SEQ_PER_CHUNK = 2
_CL = _SEQ_PER_CHUNK * _L
_NBUF = 3
_DEPTH = 2


def _emb_body(tok_hbm, idx_hbm, pos_hbm, out_hbm,
              idx_all, pos_v, b0, b1, b2,
              g0, g1, g2, o0, o1, o2,
              *, seqs_per_w, num_cores):
    bufs = (b0, b1, b2)
    gsems = (g0, g1, g2)
    osems = (o0, o1, o2)
    nrows = seqs_per_w * _L
    nchunks = nrows // _CL
    wid = lax.axis_index("s") * num_cores + lax.axis_index("c")
    base = pl.multiple_of(wid * nrows, _CL)

    pltpu.sync_copy(pos_hbm, pos_v)
    pltpu.sync_copy(idx_hbm.at[pl.ds(base, nrows)], idx_all)

    def start_gather(c, b):
        off = pl.multiple_of(c * _CL, 8)
        pltpu.async_copy(tok_hbm.at[idx_all.at[pl.ds(off, _CL)]],
                         bufs[b], gsems[b])

    def drain_gather(b):
        pltpu.make_async_copy(tok_hbm.at[pl.ds(0, _CL)], bufs[b],
                              gsems[b]).wait()

    def start_out(c, b):
        off = pl.multiple_of(base + c * _CL, _CL)
        pltpu.async_copy(bufs[b],
                         out_hbm.at[pl.ds(off, _CL), pl.ds(0, _D)],
                         osems[b])

    def drain_out(b):
        pltpu.make_async_copy(bufs[b], out_hbm.at[pl.ds(0, _CL), pl.ds(0, _D)],
                              osems[b]).wait()

    def add_pos(b):
        def add_row(l, c2):
            buf = bufs[b]
            for k in range(_D // _LANES):
                sl = pl.ds(k * _LANES, _LANES)
                p = pos_v[l, sl]
                for s in range(_SEQ_PER_CHUNK):
                    r = s * _L + l
                    buf[r, sl] = buf[r, sl] + p
            return c2
        lax.fori_loop(0, _L, add_row, 0, unroll=2)

    def process(c, b):
        drain_gather(b)
        add_pos(b)
        start_out(c, b)
        b2_ = (b + _DEPTH) % _NBUF

        @pl.when(c + _DEPTH < nchunks)
        def _():
            @pl.when(c >= 1)
            def _():
                drain_out(b2_)
            start_gather(c + _DEPTH, b2_)

    for b in range(_DEPTH):
        start_gather(b, b)

    def wave(w, carry):
        for b in range(_NBUF):
            process(w * _NBUF + b, b)
        return carry

    lax.fori_loop(0, (nchunks - 1) // _NBUF, wave, 0)
    process(nchunks - 1, (nchunks - 1) % _NBUF)
    for b in range(_NBUF):
        drain_out(b)


_TBLK = 8192


def _tpad_body(tT_ref, out_ref):
    r = tT_ref[...].T
    out_ref[...] = jnp.concatenate([r, r], axis=1)


def kernel(x, token_table, pos_table):
    B, L = x.shape
    V, D = token_table.shape
    assert L == _L and D == _D
    info = plsc.get_sparse_core_info()
    nw = info.num_cores * info.num_subcores
    assert B % nw == 0
    seqs_per_w = B // nw

    mesh = plsc.VectorSubcoreMesh(core_axis_name="c", subcore_axis_name="s")
    nblk = (V + _TBLK - 1) // _TBLK
    tok_pad = pl.pallas_call(
        _tpad_body,
        grid=(nblk,),
        in_specs=[pl.BlockSpec((_D, _TBLK), lambda i: (0, i))],
        out_specs=pl.BlockSpec((_TBLK, 2 * _D), lambda i: (i, 0)),
        out_shape=jax.ShapeDtypeStruct((V, 2 * D), jnp.float32),
    )(token_table.T)
    tok2 = tok_pad.reshape(2 * V, D)
    flat_idx = x.reshape(B * L).astype(jnp.int32) * 2

    sems = [pltpu.SemaphoreType.DMA] * (2 * _NBUF)
    emb = functools.partial(
        pl.kernel,
        mesh=mesh,
        out_type=jax.ShapeDtypeStruct((B * L, 2 * D), jnp.float32),
        scratch_types=[
            pltpu.VMEM((seqs_per_w * _L,), jnp.int32),
            pltpu.VMEM((_L, _D), jnp.float32),
        ] + [pltpu.VMEM((_CL, _D), jnp.float32) for _ in range(_NBUF)]
          + sems,
        compiler_params=pltpu.CompilerParams(use_tc_tiling_on_sc=False),
    )(functools.partial(_emb_body, seqs_per_w=seqs_per_w,
                        num_cores=info.num_cores))

    out = emb(tok2, flat_idx, pos_table)
    return out[:, :_D].reshape(B, L, D)


# speedup vs baseline: 2.3068x; 1.0448x over previous
import functools

import jax
import jax.numpy as jnp
from jax import lax
from jax.experimental import pallas as pl
from jax.experimental.pallas import tpu as pltpu
from jax.experimental.pallas import tpu_sc as plsc

_L = 200
_D = 64
_LANES = 16
_SEQ_PER_CHUNK = 2
_CL = _SEQ_PER_CHUNK * _L
_NBUF = 3
_DEPTH = 2


def _emb_body(tok_hbm, idx_hbm, pos_hbm, out_hbm,
              idx_all, pos_v, b0, b1, b2,
              g0, g1, g2, o0, o1, o2,
              *, seqs_per_w, num_cores):
    bufs = (b0, b1, b2)
    gsems = (g0, g1, g2)
    osems = (o0, o1, o2)
    nrows = seqs_per_w * _L
    nchunks = nrows // _CL
    wid = lax.axis_index("s") * num_cores + lax.axis_index("c")
    base = pl.multiple_of(wid * nrows, _CL)

    pltpu.sync_copy(pos_hbm, pos_v)
    pltpu.sync_copy(idx_hbm.at[pl.ds(base, nrows)], idx_all)

    def start_gather(c, b):
        off = pl.multiple_of(c * _CL, 8)
        pltpu.async_copy(tok_hbm.at[idx_all.at[pl.ds(off, _CL)]],
                         bufs[b], gsems[b])

    def drain_gather(b):
        pltpu.make_async_copy(tok_hbm.at[pl.ds(0, _CL)], bufs[b],
                              gsems[b]).wait()

    def start_out(c, b):
        off = pl.multiple_of(base + c * _CL, _CL)
        pltpu.async_copy(bufs[b],
                         out_hbm.at[pl.ds(off, _CL), pl.ds(0, _D)],
                         osems[b])

    def drain_out(b):
        pltpu.make_async_copy(bufs[b], out_hbm.at[pl.ds(0, _CL), pl.ds(0, _D)],
                              osems[b]).wait()

    def add_pos(b):
        def add_row(l, c2):
            buf = bufs[b]
            for k in range(_D // _LANES):
                sl = pl.ds(k * _LANES, _LANES)
                p = pos_v[l, sl]
                for s in range(_SEQ_PER_CHUNK):
                    r = s * _L + l
                    buf[r, sl] = buf[r, sl] + p
            return c2
        lax.fori_loop(0, _L, add_row, 0, unroll=2)

    def process(c, b):
        drain_gather(b)
        add_pos(b)
        start_out(c, b)
        b2_ = (b + _DEPTH) % _NBUF

        @pl.when(c + _DEPTH < nchunks)
        def _():
            @pl.when(c >= 1)
            def _():
                drain_out(b2_)
            start_gather(c + _DEPTH, b2_)

    for b in range(_DEPTH):
        start_gather(b, b)

    def wave(w, carry):
        for b in range(_NBUF):
            process(w * _NBUF + b, b)
        return carry

    lax.fori_loop(0, (nchunks - 1) // _NBUF, wave, 0)
    process(nchunks - 1, (nchunks - 1) % _NBUF)
    for b in range(_NBUF):
        drain_out(b)


_TBLK = 16384


def _tpad_body(tT_ref, out_ref):
    r = tT_ref[...].T
    out_ref[...] = jnp.concatenate([r, r], axis=1)


def kernel(x, token_table, pos_table):
    B, L = x.shape
    V, D = token_table.shape
    assert L == _L and D == _D
    info = plsc.get_sparse_core_info()
    nw = info.num_cores * info.num_subcores
    assert B % nw == 0
    seqs_per_w = B // nw

    mesh = plsc.VectorSubcoreMesh(core_axis_name="c", subcore_axis_name="s")
    nblk = (V + _TBLK - 1) // _TBLK
    tok_pad = pl.pallas_call(
        _tpad_body,
        grid=(nblk,),
        in_specs=[pl.BlockSpec((_D, _TBLK), lambda i: (0, i))],
        out_specs=pl.BlockSpec((_TBLK, 2 * _D), lambda i: (i, 0)),
        out_shape=jax.ShapeDtypeStruct((V, 2 * D), jnp.float32),
    )(token_table.T)
    tok2 = tok_pad.reshape(2 * V, D)
    flat_idx = x.reshape(B * L).astype(jnp.int32) * 2

    sems = [pltpu.SemaphoreType.DMA] * (2 * _NBUF)
    emb = functools.partial(
        pl.kernel,
        mesh=mesh,
        out_type=jax.ShapeDtypeStruct((B * L, 2 * D), jnp.float32),
        scratch_types=[
            pltpu.VMEM((seqs_per_w * _L,), jnp.int32),
            pltpu.VMEM((_L, _D), jnp.float32),
        ] + [pltpu.VMEM((_CL, _D), jnp.float32) for _ in range(_NBUF)]
          + sems,
        compiler_params=pltpu.CompilerParams(use_tc_tiling_on_sc=False),
    )(functools.partial(_emb_body, seqs_per_w=seqs_per_w,
                        num_cores=info.num_cores))

    out = emb(tok2, flat_idx, pos_table)
    return out[:, :_D].reshape(B, L, D)
